# baseline (speedup 1.0000x reference)
import functools

import jax
import jax.numpy as jnp
from jax import lax
from jax.experimental import pallas as pl
from jax.experimental.pallas import tpu as pltpu
from jax.experimental.pallas import tpu_sc as plsc

N = 10000
E = 160000
H = 128
NB = 8
SUB = H // NB
R = 8

NC = 2
NS = 16
NCRUN = 2
NW = NCRUN * NS
EPT = E // NW
C = 64
MB = 1024
NCHB = MB // C
NBLK = -(-EPT // MB)
EPT_PAD = NBLK * MB
E_PAD = (NW - 1) * EPT + EPT_PAD
ZR = 40
ROWS_PER_WR = 1000

_mesh = plsc.VectorSubcoreMesh(
    core_axis_name="c", subcore_axis_name="s", num_cores=NCRUN, num_subcores=NS
)


@functools.partial(
    pl.kernel,
    out_type=jax.ShapeDtypeStruct((NCRUN, N, H), jnp.float32),
    mesh=_mesh,
    scratch_types=[
        pltpu.VMEM_SHARED((N, H), jnp.float32),
        pltpu.VMEM((MB,), jnp.int32),
        pltpu.VMEM((MB,), jnp.int32),
        pltpu.VMEM((MB,), jnp.int32),
        pltpu.VMEM((MB,), jnp.float32),
        pltpu.VMEM((1, C), jnp.int32),
        pltpu.VMEM((1, C), jnp.int32),
        pltpu.VMEM((C, H), jnp.float32),
        pltpu.VMEM((ZR, H), jnp.float32),
        pltpu.SemaphoreType.DMA,
    ],
)
def _sc_edge_pass(xt, srch, dsth, eth, normh, a_out,
                  acc, srcv, dstv, etv, normv,
                  idx_a, seg_a, rows_a, zbuf, sem_a):
    c = lax.axis_index("c")
    s = lax.axis_index("s")
    t = c * NS + s
    e0 = t * EPT

    def _zb(i, carry):
        for q in range(H // SUB):
            zbuf[i, pl.ds(q * SUB, SUB)] = jnp.zeros((SUB,), jnp.float32)
        return carry
    lax.fori_loop(0, ZR, _zb, 0)

    lanes = lax.iota(jnp.int32, SUB)

    @pl.when(s < N // ROWS_PER_WR)
    def _clear():
        for k in range(ROWS_PER_WR // ZR):
            pltpu.sync_copy(zbuf, acc.at[pl.ds(s * ROWS_PER_WR + k * ZR, ZR)])

    plsc.subcore_barrier()

    def _blk(blk, carry):
        b0 = e0 + blk * MB
        pltpu.sync_copy(srch.at[pl.ds(b0, MB)], srcv)
        pltpu.sync_copy(dsth.at[pl.ds(b0, MB)], dstv)
        pltpu.sync_copy(eth.at[pl.ds(b0, MB)], etv)
        pltpu.sync_copy(normh.at[pl.ds(b0, MB)], normv)

        def _chunk(j, carry2):
            for g in range(C // SUB):
                loff = j * C + g * SUB
                ok = lanes < (EPT - blk * MB - loff)
                sv = srcv[pl.ds(loff, SUB)]
                ev = etv[pl.ds(loff, SUB)]
                dv = dstv[pl.ds(loff, SUB)]
                spread = t * C + g * SUB + lanes
                idx_a[0, pl.ds(g * SUB, SUB)] = jnp.where(ok, ev * N + sv,
                                                          spread)
                seg_a[0, pl.ds(g * SUB, SUB)] = jnp.where(ok, dv, spread)
            pltpu.async_copy(xt.at[idx_a.at[0]], rows_a, sem_a).wait()
            for g in range(C // SUB):
                loff = j * C + g * SUB
                ok = lanes < (EPT - blk * MB - loff)
                nv = jnp.where(ok, normv[pl.ds(loff, SUB)], 0.0)
                for e in range(SUB):
                    r = g * SUB + e
                    sc = nv[e]
                    for q in range(H // SUB):
                        rows_a[r, pl.ds(q * SUB, SUB)] = (
                            rows_a[r, pl.ds(q * SUB, SUB)] * sc
                        )
            pltpu.sync_copy(rows_a, acc.at[seg_a.at[0]], add=True)
            return carry2
        lax.fori_loop(0, NCHB, _chunk, 0)
        return carry
    lax.fori_loop(0, NBLK, _blk, 0)

    plsc.subcore_barrier()

    @pl.when(s < N // ROWS_PER_WR)
    def _writeout():
        pltpu.sync_copy(acc.at[pl.ds(s * ROWS_PER_WR, ROWS_PER_WR)],
                        a_out.at[c, pl.ds(s * ROWS_PER_WR, ROWS_PER_WR)])


NT = 400


def _tc_pre_body(x_ref, w_ref, o_ref):
    w3 = w_ref[0]
    eye = jnp.eye(NB, dtype=jnp.float32)
    wbig = (w3[:, :, None, :] * eye[:, None, :, None]).reshape(H, H)
    o_ref[...] = jnp.dot(x_ref[...], wbig,
                         preferred_element_type=jnp.float32)


_tc_pretransform = pl.pallas_call(
    _tc_pre_body,
    grid=(R,),
    in_specs=[
        pl.BlockSpec((N, H), lambda r: (0, 0)),
        pl.BlockSpec((1, NB, SUB, SUB), lambda r: (r, 0, 0, 0)),
    ],
    out_specs=pl.BlockSpec((N, H), lambda r: (r, 0)),
    out_shape=jax.ShapeDtypeStruct((R * N, H), jnp.float32),
)


def _tc_comb_body(a_ref, x_ref, lw_ref, bias_ref, o_ref):
    acc = jnp.dot(x_ref[...], lw_ref[...], preferred_element_type=jnp.float32)
    acc = acc + bias_ref[...]
    for cc in range(NCRUN):
        acc = acc + a_ref[cc]
    o_ref[...] = acc


_tc_combine = pl.pallas_call(
    _tc_comb_body,
    grid=(N // NT,),
    in_specs=[
        pl.BlockSpec((NCRUN, NT, H), lambda i: (0, i, 0)),
        pl.BlockSpec((NT, H), lambda i: (i, 0)),
        pl.BlockSpec((H, H), lambda i: (0, 0)),
        pl.BlockSpec((1, H), lambda i: (0, 0)),
    ],
    out_specs=pl.BlockSpec((NT, H), lambda i: (i, 0)),
    out_shape=jax.ShapeDtypeStruct((N, H), jnp.float32),
)


def kernel(x, edge_index, etype, norm, weight, loop_weight, h_bias):
    pad = E_PAD - E
    src = jnp.pad(edge_index[0], (0, pad))
    dst = jnp.pad(edge_index[1], (0, pad))
    etp = jnp.pad(etype, (0, pad))
    normf = jnp.pad(norm.reshape(E), (0, pad))

    xt = _tc_pretransform(x, weight)
    a = _sc_edge_pass(xt, src, dst, etp, normf)
    return _tc_combine(a, x, loop_weight, h_bias.reshape(1, H))


# speedup vs baseline: 1.3470x; 1.3470x over previous
import functools

import jax
import jax.numpy as jnp
from jax import lax
from jax.experimental import pallas as pl
from jax.experimental.pallas import tpu as pltpu
from jax.experimental.pallas import tpu_sc as plsc

N = 10000
E = 160000
H = 128
NB = 8
SUB = H // NB
R = 8

NC = 2
NS = 16
NCRUN = 2
NW = NCRUN * NS
EPT = E // NW
C = 64
MB = 1024
NCHB = MB // C
NBLK = -(-EPT // MB)
EPT_PAD = NBLK * MB
E_PAD = (NW - 1) * EPT + EPT_PAD
ZR = 40
ROWS_PER_WR = 1000

_mesh = plsc.VectorSubcoreMesh(
    core_axis_name="c", subcore_axis_name="s", num_cores=NCRUN, num_subcores=NS
)


@functools.partial(
    pl.kernel,
    out_type=jax.ShapeDtypeStruct((NCRUN, N, H), jnp.float32),
    mesh=_mesh,
    scratch_types=[
        pltpu.VMEM_SHARED((N, H), jnp.float32),
        pltpu.VMEM((MB,), jnp.int32),
        pltpu.VMEM((MB,), jnp.int32),
        pltpu.VMEM((MB,), jnp.int32),
        pltpu.VMEM((MB,), jnp.float32),
        pltpu.VMEM((1, C), jnp.int32),
        pltpu.VMEM((1, C), jnp.int32),
        pltpu.VMEM((1, C), jnp.int32),
        pltpu.VMEM((1, C), jnp.int32),
        pltpu.VMEM((C, H), jnp.float32),
        pltpu.VMEM((C, H), jnp.float32),
        pltpu.VMEM((ZR, H), jnp.float32),
        pltpu.SemaphoreType.DMA,
        pltpu.SemaphoreType.DMA,
    ],
)
def _sc_edge_pass(xt, srch, dsth, eth, normh, a_out,
                  acc, srcv, dstv, etv, normv,
                  idx_a, seg_a, idx_b, seg_b, rows_a, rows_b, zbuf,
                  sem_a, sem_b):
    c = lax.axis_index("c")
    s = lax.axis_index("s")
    t = c * NS + s
    e0 = t * EPT

    def _zb(i, carry):
        for q in range(H // SUB):
            zbuf[i, pl.ds(q * SUB, SUB)] = jnp.zeros((SUB,), jnp.float32)
        return carry
    lax.fori_loop(0, ZR, _zb, 0)

    lanes = lax.iota(jnp.int32, SUB)

    @pl.when(s < N // ROWS_PER_WR)
    def _clear():
        for k in range(ROWS_PER_WR // ZR):
            pltpu.sync_copy(zbuf, acc.at[pl.ds(s * ROWS_PER_WR + k * ZR, ZR)])

    plsc.subcore_barrier()

    def _build(blk, j, idx1, seg1):
        for g in range(C // SUB):
            loff = j * C + g * SUB
            ok = lanes < (EPT - blk * MB - loff)
            sv = srcv[pl.ds(loff, SUB)]
            ev = etv[pl.ds(loff, SUB)]
            dv = dstv[pl.ds(loff, SUB)]
            spread = t * C + g * SUB + lanes
            idx1[0, pl.ds(g * SUB, SUB)] = jnp.where(ok, ev * N + sv, spread)
            seg1[0, pl.ds(g * SUB, SUB)] = jnp.where(ok, dv, spread)

    def _scale(blk, j, rows):
        for g in range(C // SUB):
            loff = j * C + g * SUB
            ok = lanes < (EPT - blk * MB - loff)
            nv = jnp.where(ok, normv[pl.ds(loff, SUB)], 0.0)
            for e in range(SUB):
                r = g * SUB + e
                sc = nv[e]
                for q in range(H // SUB):
                    rows[r, pl.ds(q * SUB, SUB)] = (
                        rows[r, pl.ds(q * SUB, SUB)] * sc
                    )

    def _blk(blk, carry):
        b0 = e0 + blk * MB
        pltpu.sync_copy(srch.at[pl.ds(b0, MB)], srcv)
        pltpu.sync_copy(dsth.at[pl.ds(b0, MB)], dstv)
        pltpu.sync_copy(eth.at[pl.ds(b0, MB)], etv)
        pltpu.sync_copy(normh.at[pl.ds(b0, MB)], normv)

        _build(blk, 0, idx_a, seg_a)
        pltpu.async_copy(xt.at[idx_a.at[0]], rows_a, sem_a)

        def _pipe(jj, carry2):
            j0 = 2 * jj
            j1 = j0 + 1
            _build(blk, j1, idx_b, seg_b)
            pltpu.async_copy(xt.at[idx_b.at[0]], rows_b, sem_b)
            pltpu.make_async_copy(xt.at[idx_a.at[0]], rows_a, sem_a).wait()
            _scale(blk, j0, rows_a)
            pltpu.sync_copy(rows_a, acc.at[seg_a.at[0]], add=True)
            jn = jnp.minimum(j0 + 2, NCHB - 1)
            _build(blk, jn, idx_a, seg_a)
            pltpu.async_copy(xt.at[idx_a.at[0]], rows_a, sem_a)
            pltpu.make_async_copy(xt.at[idx_b.at[0]], rows_b, sem_b).wait()
            _scale(blk, j1, rows_b)
            pltpu.sync_copy(rows_b, acc.at[seg_b.at[0]], add=True)
            return carry2
        lax.fori_loop(0, NCHB // 2, _pipe, 0)

        pltpu.make_async_copy(xt.at[idx_a.at[0]], rows_a, sem_a).wait()
        return carry
    lax.fori_loop(0, NBLK, _blk, 0)

    plsc.subcore_barrier()

    @pl.when(s < N // ROWS_PER_WR)
    def _writeout():
        pltpu.sync_copy(acc.at[pl.ds(s * ROWS_PER_WR, ROWS_PER_WR)],
                        a_out.at[c, pl.ds(s * ROWS_PER_WR, ROWS_PER_WR)])


NT = 400


def _tc_pre_body(x_ref, w_ref, o_ref):
    w3 = w_ref[0]
    eye = jnp.eye(NB, dtype=jnp.float32)
    wbig = (w3[:, :, None, :] * eye[:, None, :, None]).reshape(H, H)
    o_ref[...] = jnp.dot(x_ref[...], wbig,
                         preferred_element_type=jnp.float32)


_tc_pretransform = pl.pallas_call(
    _tc_pre_body,
    grid=(R,),
    in_specs=[
        pl.BlockSpec((N, H), lambda r: (0, 0)),
        pl.BlockSpec((1, NB, SUB, SUB), lambda r: (r, 0, 0, 0)),
    ],
    out_specs=pl.BlockSpec((N, H), lambda r: (r, 0)),
    out_shape=jax.ShapeDtypeStruct((R * N, H), jnp.float32),
)


def _tc_comb_body(a_ref, x_ref, lw_ref, bias_ref, o_ref):
    acc = jnp.dot(x_ref[...], lw_ref[...], preferred_element_type=jnp.float32)
    acc = acc + bias_ref[...]
    for cc in range(NCRUN):
        acc = acc + a_ref[cc]
    o_ref[...] = acc


_tc_combine = pl.pallas_call(
    _tc_comb_body,
    grid=(N // NT,),
    in_specs=[
        pl.BlockSpec((NCRUN, NT, H), lambda i: (0, i, 0)),
        pl.BlockSpec((NT, H), lambda i: (i, 0)),
        pl.BlockSpec((H, H), lambda i: (0, 0)),
        pl.BlockSpec((1, H), lambda i: (0, 0)),
    ],
    out_specs=pl.BlockSpec((NT, H), lambda i: (i, 0)),
    out_shape=jax.ShapeDtypeStruct((N, H), jnp.float32),
)


def kernel(x, edge_index, etype, norm, weight, loop_weight, h_bias):
    pad = E_PAD - E
    src = jnp.pad(edge_index[0], (0, pad))
    dst = jnp.pad(edge_index[1], (0, pad))
    etp = jnp.pad(etype, (0, pad))
    normf = jnp.pad(norm.reshape(E), (0, pad))

    xt = _tc_pretransform(x, weight)
    a = _sc_edge_pass(xt, src, dst, etp, normf)
    return _tc_combine(a, x, loop_weight, h_bias.reshape(1, H))
